# baseline (speedup 1.0000x reference)
import functools

import jax
import jax.numpy as jnp
from jax.experimental import pallas as pl
from jax.experimental.pallas import tpu as pltpu


def _transform_kernel(xT_ref, U0_ref, b0_ref, out_ref):
    out_ref[...] = jax.lax.dot_general(
        U0_ref[...], xT_ref[...], (((0,), (0,)), ((), ())),
        preferred_element_type=jnp.float32) + b0_ref[...]


def _transform(xT, U0, b0c):
    D, N = xT.shape
    bn = 512
    return pl.pallas_call(
        _transform_kernel,
        grid=(N // bn,),
        in_specs=[
            pl.BlockSpec((D, bn), lambda j: (0, j)),
            pl.BlockSpec((D, D), lambda j: (0, 0)),
            pl.BlockSpec((D, 1), lambda j: (0, 0)),
        ],
        out_specs=pl.BlockSpec((D, bn), lambda j: (0, j)),
        out_shape=jax.ShapeDtypeStruct((D, N), jnp.float32),
    )(xT, U0, b0c)


def _layer_kernel(nk, y_ref, A_ref, U_ref, out_ref, acc_ref, deg_ref):
    k = pl.program_id(1)

    @pl.when(k == 0)
    def _init():
        acc_ref[...] = jnp.zeros_like(acc_ref)
        deg_ref[...] = jnp.zeros_like(deg_ref)

    Af = A_ref[...].astype(jnp.float32)
    acc_ref[...] += jnp.dot(y_ref[...], Af, preferred_element_type=jnp.float32)
    deg_ref[...] += jnp.sum(Af, axis=0, keepdims=True)

    @pl.when(k == nk - 1)
    def _epilogue():
        agg = acc_ref[...] / deg_ref[...]
        out_ref[...] = jnp.maximum(
            jnp.dot(U_ref[...], agg, preferred_element_type=jnp.float32), 0.0)


def _layer(y, adj, U, bi=512, bk=1024):
    D, N = y.shape
    ni, nk = N // bi, N // bk
    return pl.pallas_call(
        functools.partial(_layer_kernel, nk),
        grid=(ni, nk),
        in_specs=[
            pl.BlockSpec((D, bk), lambda i, k: (0, k)),
            pl.BlockSpec((bk, bi), lambda i, k: (k, i)),
            pl.BlockSpec((D, D), lambda i, k: (0, 0)),
        ],
        out_specs=pl.BlockSpec((D, bi), lambda i, k: (0, i)),
        out_shape=jax.ShapeDtypeStruct((D, N), jnp.float32),
        scratch_shapes=[
            pltpu.VMEM((D, bi), jnp.float32),
            pltpu.VMEM((1, bi), jnp.float32),
        ],
        compiler_params=pltpu.CompilerParams(
            dimension_semantics=("parallel", "arbitrary")),
    )(y, adj, U)


def kernel(x, adj_mat, U0, b0, U1, U2):
    N, D = x.shape
    xT = x.T
    y0 = _transform(xT, U0, b0.reshape(D, 1))
    y1 = _layer(y0, adj_mat, U1)
    y2 = _layer(y1, adj_mat, U2)
    return y2.T
